# baseline (speedup 1.0000x reference)
import functools

import jax
import jax.numpy as jnp
from jax import lax
from jax.experimental import pallas as pl
from jax.experimental.pallas import tpu as pltpu
from jax.experimental.pallas import tpu_sc as plsc

_N_NODES = 10000
_N_PAD = 10240
_N_EDGES = 320000
_DIM = 128
_NC = 2
_NS = 16
_NW = _NC * _NS
_K = 80
_CPW = 128
_E_PAD = _NW * _CPW * _K
_RPT = _N_PAD // _NS
_ZR = _K
_QC = 4


def _sc_aggregate(x, idx2d):
    mesh = plsc.VectorSubcoreMesh(core_axis_name="c", subcore_axis_name="s")

    @functools.partial(
        pl.kernel,
        out_type=jax.ShapeDtypeStruct((_NC, _N_PAD, _DIM), jnp.float32),
        mesh=mesh,
        scratch_types=[
            pltpu.VMEM_SHARED((_N_PAD, _DIM), jnp.float32),
            pltpu.VMEM((_K, _DIM), jnp.float32),
            pltpu.VMEM((2 * _QC, _K), jnp.int32),
            pltpu.SemaphoreType.DMA,
        ],
    )
    def k(x_hbm, idx_hbm, out_hbm, accum, rows, iq, gsem):
        c = lax.axis_index("c")
        s = lax.axis_index("s")
        w = s * _NC + c
        q0 = w * (_CPW // _QC)

        def zero_row(r, carry):
            for j in range(_DIM // 16):
                rows[r, pl.ds(j * 16, 16)] = jnp.zeros((16,), jnp.float32)
            return carry

        lax.fori_loop(0, _ZR, zero_row, 0)
        row0 = s * _RPT
        for j in range(_RPT // _ZR):
            pltpu.sync_copy(rows, accum.at[pl.ds(row0 + j * _ZR, _ZR)])
        plsc.subcore_barrier()

        def quad(q, carry):
            pltpu.sync_copy(
                idx_hbm.at[pl.ds((q0 + q) * 2 * _QC, 2 * _QC)], iq)
            for t in range(_QC):
                pltpu.async_copy(x_hbm.at[iq.at[2 * t]], rows, gsem).wait()
                pltpu.sync_copy(rows, accum.at[iq.at[2 * t + 1]], add=True)
            return carry

        lax.fori_loop(0, _CPW // _QC, quad, 0)
        plsc.subcore_barrier()

        for j in range(_RPT // _ZR):
            r = row0 + j * _ZR
            pltpu.sync_copy(accum.at[pl.ds(r, _ZR)], rows)
            pltpu.sync_copy(rows, out_hbm.at[c].at[pl.ds(r, _ZR)])

    return k(x, idx2d)


def _mm_relu(partials, W):
    blk = 1024

    def body(p0_ref, p1_ref, w_ref, o_ref):
        ssum = p0_ref[...] + p1_ref[...]
        o_ref[...] = jnp.maximum(
            jnp.dot(ssum, w_ref[...], preferred_element_type=jnp.float32),
            0.0)

    return pl.pallas_call(
        body,
        grid=(_N_PAD // blk,),
        in_specs=[
            pl.BlockSpec((blk, _DIM), lambda i: (i, 0)),
            pl.BlockSpec((blk, _DIM), lambda i: (i, 0)),
            pl.BlockSpec((_DIM, _DIM), lambda i: (0, 0)),
        ],
        out_specs=pl.BlockSpec((blk, _DIM), lambda i: (i, 0)),
        out_shape=jax.ShapeDtypeStruct((_N_NODES, _DIM), jnp.float32),
    )(partials[0], partials[1], W)


def kernel(x, edge_index, W):
    src = edge_index[1].astype(jnp.int32)
    dst = edge_index[0].astype(jnp.int32)
    npad = _E_PAD - _N_EDGES
    src_p = jnp.concatenate(
        [src, jnp.zeros((npad,), jnp.int32)]).reshape(-1, _K)
    dst_p = jnp.concatenate(
        [dst, jnp.full((npad,), _N_NODES, jnp.int32)]).reshape(-1, _K)
    idx2d = jnp.stack([src_p, dst_p], axis=1).reshape(-1, _K)
    partials = _sc_aggregate(x, idx2d)
    return _mm_relu(partials, W)


# speedup vs baseline: 1.2179x; 1.2179x over previous
import functools

import jax
import jax.numpy as jnp
from jax import lax
from jax.experimental import pallas as pl
from jax.experimental.pallas import tpu as pltpu
from jax.experimental.pallas import tpu_sc as plsc

_N_NODES = 10000
_N_PAD = 10240
_N_EDGES = 320000
_DIM = 128
_NC = 2
_NS = 16
_NW = _NC * _NS
_K = 80
_CPW = 128
_E_PAD = _NW * _CPW * _K
_RPT = _N_PAD // _NS
_ZR = _K


def _sc_aggregate(x, src_p, dst_p):
    mesh = plsc.VectorSubcoreMesh(core_axis_name="c", subcore_axis_name="s")

    @functools.partial(
        pl.kernel,
        out_type=jax.ShapeDtypeStruct((_NC, _N_PAD, _DIM), jnp.float32),
        mesh=mesh,
        scratch_types=[
            pltpu.VMEM_SHARED((_N_PAD, _DIM), jnp.float32),
            [pltpu.VMEM((_K, _DIM), jnp.float32)] * 2,
            [pltpu.VMEM((_K,), jnp.int32)] * 2,
            pltpu.VMEM((_K,), jnp.int32),
            [pltpu.SemaphoreType.DMA] * 2,
        ],
    )
    def k(x_hbm, src_hbm, dst_hbm, out_hbm, accum, ring, sidx, dbuf, gsem):
        c = lax.axis_index("c")
        s = lax.axis_index("s")
        w = s * _NC + c
        e0 = w * _CPW * _K

        def zero_row(r, carry):
            for j in range(_DIM // 16):
                ring[0][r, pl.ds(j * 16, 16)] = jnp.zeros((16,), jnp.float32)
            return carry

        lax.fori_loop(0, _ZR, zero_row, 0)
        row0 = s * _RPT
        for j in range(_RPT // _ZR):
            pltpu.sync_copy(ring[0], accum.at[pl.ds(row0 + j * _ZR, _ZR)])
        plsc.subcore_barrier()

        def sload(j, p):
            pltpu.sync_copy(src_hbm.at[pl.ds(e0 + j * _K, _K)], sidx[p])

        def gstart(p):
            pltpu.async_copy(x_hbm.at[sidx[p]], ring[p], gsem[p])

        def gwait(p):
            pltpu.make_async_copy(x_hbm.at[sidx[p]], ring[p], gsem[p]).wait()

        def drain(j, p):
            pltpu.sync_copy(dst_hbm.at[pl.ds(e0 + j * _K, _K)], dbuf)
            gwait(p)
            pltpu.sync_copy(ring[p], accum.at[dbuf], add=True)

        sload(0, 0)
        gstart(0)

        def pair(i, carry):
            j0 = 2 * i
            sload(j0 + 1, 1)
            gstart(1)
            drain(j0, 0)
            sload(j0 + 2, 0)
            gstart(0)
            drain(j0 + 1, 1)
            return carry

        lax.fori_loop(0, _CPW // 2 - 1, pair, 0)
        sload(_CPW - 1, 1)
        gstart(1)
        drain(_CPW - 2, 0)
        drain(_CPW - 1, 1)
        plsc.subcore_barrier()

        for j in range(_RPT // _ZR):
            r = row0 + j * _ZR
            pltpu.sync_copy(accum.at[pl.ds(r, _ZR)], ring[0])
            pltpu.sync_copy(ring[0], out_hbm.at[c].at[pl.ds(r, _ZR)])

    return k(x, src_p, dst_p)


def _mm_relu(partials, W):
    blk = 1024

    def body(p0_ref, p1_ref, w_ref, o_ref):
        ssum = p0_ref[...] + p1_ref[...]
        o_ref[...] = jnp.maximum(
            jnp.dot(ssum, w_ref[...], preferred_element_type=jnp.float32),
            0.0)

    return pl.pallas_call(
        body,
        grid=(_N_PAD // blk,),
        in_specs=[
            pl.BlockSpec((blk, _DIM), lambda i: (i, 0)),
            pl.BlockSpec((blk, _DIM), lambda i: (i, 0)),
            pl.BlockSpec((_DIM, _DIM), lambda i: (0, 0)),
        ],
        out_specs=pl.BlockSpec((blk, _DIM), lambda i: (i, 0)),
        out_shape=jax.ShapeDtypeStruct((_N_NODES, _DIM), jnp.float32),
    )(partials[0], partials[1], W)


def kernel(x, edge_index, W):
    src = edge_index[1].astype(jnp.int32)
    dst = edge_index[0].astype(jnp.int32)
    npad = _E_PAD - _N_EDGES
    src_p = jnp.concatenate([src, jnp.zeros((npad,), jnp.int32)])
    dst_p = jnp.concatenate([dst, jnp.full((npad,), _N_NODES, jnp.int32)])
    partials = _sc_aggregate(x, src_p, dst_p)
    return _mm_relu(partials, W)
